# baseline (speedup 1.0000x reference)
import jax
import jax.numpy as jnp
from jax.experimental import pallas as pl
from jax.experimental.pallas import tpu as pltpu


def _clocs_kernel(q_ref, b_ref, out_ref, idx_ref, cnt_ref):
    i = pl.program_id(0)
    bk, L = out_ref.shape
    L2 = idx_ref.shape[1]

    bx1 = b_ref[0:1, :]
    by1 = b_ref[1:2, :]
    bx2 = b_ref[2:3, :]
    by2 = b_ref[3:4, :]
    ab = b_ref[4:5, :]
    base = b_ref[5:6, :]

    qx1 = q_ref[:, 0:1]
    qy1 = q_ref[:, 1:2]
    qx2 = q_ref[:, 2:3]
    qy2 = q_ref[:, 3:4]
    aq = q_ref[:, 4:5]
    s2 = q_ref[:, 5:6]

    iw = jnp.minimum(bx2, qx2) - jnp.maximum(bx1, qx1)
    ih = jnp.minimum(by2, qy2) - jnp.maximum(by1, qy1)
    inter = iw * ih
    iou = inter / (ab + aq - inter)
    valid = jnp.minimum(iw, ih) > 0.0

    lane = jax.lax.broadcasted_iota(jnp.int32, (bk, L), 1)
    j = lane & 3
    f0 = jnp.where(valid, iou, -10.0)
    f2 = jnp.where(valid, s2, -10.0)
    out_ref[...] = jnp.where(j == 0, f0, jnp.where(j == 2, f2, base))

    lane2 = jax.lax.broadcasted_iota(jnp.int32, (bk, L2), 1)
    row2 = jax.lax.broadcasted_iota(jnp.int32, (bk, L2), 0)
    idx_ref[...] = jnp.where((lane2 & 1) == 0, i * bk + row2, lane2 >> 1)

    c = jnp.sum((valid & (j == 0)).astype(jnp.int32))

    @pl.when(i == 0)
    def _init():
        cnt_ref[0, 0] = 0

    cnt_ref[0, 0] += c


def kernel(boxes, query_boxes, scores_3d, scores_2d, dis_to_lidar_3d):
    n = boxes.shape[0]
    k = query_boxes.shape[0]
    bk = 8

    b = boxes
    area_b = (b[:, 2] - b[:, 0]) * (b[:, 3] - b[:, 1])
    zeros = jnp.zeros((n,), jnp.float32)
    rep = jnp.stack([b[:, 0], b[:, 1], b[:, 2], b[:, 3], area_b], 0)
    rep = jnp.repeat(rep, 4, axis=1)
    base = jnp.stack(
        [zeros, scores_3d[:, 0], zeros, dis_to_lidar_3d[:, 0]], 1
    ).reshape(1, 4 * n)
    bdata = jnp.concatenate([rep, base, jnp.zeros((2, 4 * n), jnp.float32)], 0)

    area_q = (query_boxes[:, 2] - query_boxes[:, 0]) * (
        query_boxes[:, 3] - query_boxes[:, 1])
    qdata = jnp.concatenate(
        [query_boxes, area_q[:, None], scores_2d, jnp.zeros((k, 2), jnp.float32)],
        axis=1)

    grid = k // bk
    out_flat, idx_flat, cnt = pl.pallas_call(
        _clocs_kernel,
        grid=(grid,),
        in_specs=[
            pl.BlockSpec((bk, 8), lambda i: (i, 0)),
            pl.BlockSpec((8, 4 * n), lambda i: (0, 0)),
        ],
        out_specs=[
            pl.BlockSpec((bk, 4 * n), lambda i: (i, 0)),
            pl.BlockSpec((bk, 2 * n), lambda i: (i, 0)),
            pl.BlockSpec(memory_space=pltpu.SMEM, block_shape=(1, 1),
                         index_map=lambda i: (0, 0)),
        ],
        out_shape=[
            jax.ShapeDtypeStruct((k, 4 * n), jnp.float32),
            jax.ShapeDtypeStruct((k, 2 * n), jnp.int32),
            jax.ShapeDtypeStruct((1, 1), jnp.int32),
        ],
    )(qdata, bdata)

    overlaps = out_flat.reshape(k, n, 4)
    tensor_index = idx_flat.reshape(k, n, 2)
    return overlaps, tensor_index, cnt[0, 0]


# speedup vs baseline: 1.5341x; 1.5341x over previous
import jax
import jax.numpy as jnp
from jax.experimental import pallas as pl
from jax.experimental.pallas import tpu as pltpu


def _clocs_kernel(q_ref, b_ref, f0_ref, f2_ref, cnt_ref):
    i = pl.program_id(0)

    bx1 = b_ref[0:1, :]
    by1 = b_ref[1:2, :]
    bx2 = b_ref[2:3, :]
    by2 = b_ref[3:4, :]
    ab = b_ref[4:5, :]

    qx1 = q_ref[:, 0:1]
    qy1 = q_ref[:, 1:2]
    qx2 = q_ref[:, 2:3]
    qy2 = q_ref[:, 3:4]
    aq = q_ref[:, 4:5]
    s2 = q_ref[:, 5:6]

    iw = jnp.minimum(bx2, qx2) - jnp.maximum(bx1, qx1)
    ih = jnp.minimum(by2, qy2) - jnp.maximum(by1, qy1)
    inter = iw * ih
    iou = inter / (ab + aq - inter)
    valid = jnp.minimum(iw, ih) > 0.0

    f0_ref[...] = jnp.where(valid, iou, -10.0)
    f2_ref[...] = jnp.where(valid, s2, -10.0)

    c = jnp.sum(valid.astype(jnp.int32))

    @pl.when(i == 0)
    def _init():
        cnt_ref[0, 0] = 0

    cnt_ref[0, 0] += c


def kernel(boxes, query_boxes, scores_3d, scores_2d, dis_to_lidar_3d):
    n = boxes.shape[0]
    k = query_boxes.shape[0]
    bk = 8

    b = boxes
    area_b = (b[:, 2] - b[:, 0]) * (b[:, 3] - b[:, 1])
    bdata = jnp.stack(
        [b[:, 0], b[:, 1], b[:, 2], b[:, 3], area_b,
         jnp.zeros((n,), jnp.float32), jnp.zeros((n,), jnp.float32),
         jnp.zeros((n,), jnp.float32)], 0)

    area_q = (query_boxes[:, 2] - query_boxes[:, 0]) * (
        query_boxes[:, 3] - query_boxes[:, 1])
    qdata = jnp.concatenate(
        [query_boxes, area_q[:, None], scores_2d, jnp.zeros((k, 2), jnp.float32)],
        axis=1)

    grid = k // bk
    f0, f2, cnt = pl.pallas_call(
        _clocs_kernel,
        grid=(grid,),
        in_specs=[
            pl.BlockSpec((bk, 8), lambda i: (i, 0)),
            pl.BlockSpec((8, n), lambda i: (0, 0)),
        ],
        out_specs=[
            pl.BlockSpec((bk, n), lambda i: (i, 0)),
            pl.BlockSpec((bk, n), lambda i: (i, 0)),
            pl.BlockSpec(memory_space=pltpu.SMEM, block_shape=(1, 1),
                         index_map=lambda i: (0, 0)),
        ],
        out_shape=[
            jax.ShapeDtypeStruct((k, n), jnp.float32),
            jax.ShapeDtypeStruct((k, n), jnp.float32),
            jax.ShapeDtypeStruct((1, 1), jnp.int32),
        ],
    )(qdata, bdata)

    f1 = jnp.broadcast_to(scores_3d[:, 0][None, :], (k, n))
    f3 = jnp.broadcast_to(dis_to_lidar_3d[:, 0][None, :], (k, n))
    overlaps = jnp.stack([f0, f1, f2, f3], axis=-1)
    kk, nn = jnp.meshgrid(jnp.arange(k, dtype=jnp.int32),
                          jnp.arange(n, dtype=jnp.int32), indexing="ij")
    tensor_index = jnp.stack([kk, nn], axis=-1)
    return overlaps, tensor_index, cnt[0, 0]


# speedup vs baseline: 5.3936x; 3.5157x over previous
import jax
import jax.numpy as jnp
from jax.experimental import pallas as pl
from jax.experimental.pallas import tpu as pltpu


def _clocs_kernel(q_ref, bx1_ref, by1_ref, bx2_ref, by2_ref, ab_ref, base_ref,
                  tib_ref, out_ref, ti_ref, cnt_ref):
    i = pl.program_id(0)
    bk = out_ref.shape[0]
    n = out_ref.shape[2]

    bx1 = bx1_ref[...]
    by1 = by1_ref[...]
    bx2 = bx2_ref[...]
    by2 = by2_ref[...]
    ab = ab_ref[...]
    base = base_ref[...]
    tib = tib_ref[...]

    row4 = jax.lax.broadcasted_iota(jnp.int32, (4, n), 0)
    is0 = row4 == 0
    is13 = (row4 & 1) == 1
    row2 = jax.lax.broadcasted_iota(jnp.int32, (2, n), 0)
    is_k_row = row2 == 0

    acc = jnp.zeros((4, n), jnp.int32)
    for kk in range(bk):
        qx1 = q_ref[kk, 0]
        qy1 = q_ref[kk, 1]
        qx2 = q_ref[kk, 2]
        qy2 = q_ref[kk, 3]
        aq = q_ref[kk, 4]
        s2 = q_ref[kk, 5]

        iw = jnp.minimum(bx2, qx2) - jnp.maximum(bx1, qx1)
        ih = jnp.minimum(by2, qy2) - jnp.maximum(by1, qy1)
        inter = iw * ih
        iou = inter / ((ab + aq) - inter)
        valid = jnp.minimum(iw, ih) > 0.0

        f02 = jnp.where(valid, jnp.where(is0, iou, s2), -10.0)
        out_ref[kk] = jnp.where(is13, base, f02)

        kg = i * bk + kk
        ti_ref[kk] = jnp.where(is_k_row, kg, tib)

        acc = acc + jnp.where(valid, 1, 0)

    c = jnp.sum(acc) >> 2

    @pl.when(i == 0)
    def _init():
        cnt_ref[0, 0] = 0

    cnt_ref[0, 0] += c


def _rep4(x):
    return jnp.broadcast_to(x[None, :], (4, x.shape[0]))


def kernel(boxes, query_boxes, scores_3d, scores_2d, dis_to_lidar_3d):
    n = boxes.shape[0]
    k = query_boxes.shape[0]
    bk = 8

    b = boxes
    area_b = (b[:, 2] - b[:, 0]) * (b[:, 3] - b[:, 1])
    bx1 = _rep4(b[:, 0])
    by1 = _rep4(b[:, 1])
    bx2 = _rep4(b[:, 2])
    by2 = _rep4(b[:, 3])
    ab = _rep4(area_b)
    zeros = jnp.zeros((n,), jnp.float32)
    base = jnp.stack([zeros, scores_3d[:, 0], zeros, dis_to_lidar_3d[:, 0]], 0)
    tib = jnp.broadcast_to(jnp.arange(n, dtype=jnp.int32)[None, :], (2, n))

    area_q = (query_boxes[:, 2] - query_boxes[:, 0]) * (
        query_boxes[:, 3] - query_boxes[:, 1])
    qdata = jnp.concatenate(
        [query_boxes, area_q[:, None], scores_2d, jnp.zeros((k, 2), jnp.float32)],
        axis=1)

    grid = k // bk
    vec = lambda: pl.BlockSpec((4, n), lambda i: (0, 0))
    out, ti, cnt = pl.pallas_call(
        _clocs_kernel,
        grid=(grid,),
        in_specs=[
            pl.BlockSpec((bk, 8), lambda i: (i, 0), memory_space=pltpu.SMEM),
            vec(), vec(), vec(), vec(), vec(), vec(),
            pl.BlockSpec((2, n), lambda i: (0, 0)),
        ],
        out_specs=[
            pl.BlockSpec((bk, 4, n), lambda i: (i, 0, 0)),
            pl.BlockSpec((bk, 2, n), lambda i: (i, 0, 0)),
            pl.BlockSpec(memory_space=pltpu.SMEM, block_shape=(1, 1),
                         index_map=lambda i: (0, 0)),
        ],
        out_shape=[
            jax.ShapeDtypeStruct((k, 4, n), jnp.float32),
            jax.ShapeDtypeStruct((k, 2, n), jnp.int32),
            jax.ShapeDtypeStruct((1, 1), jnp.int32),
        ],
    )(qdata, bx1, by1, bx2, by2, ab, base, tib)

    overlaps = jnp.transpose(out, (0, 2, 1))
    tensor_index = jnp.transpose(ti, (0, 2, 1))
    return overlaps, tensor_index, cnt[0, 0]


# speedup vs baseline: 8.1370x; 1.5087x over previous
import jax
import jax.numpy as jnp
from jax.experimental import pallas as pl
from jax.experimental.pallas import tpu as pltpu


def _clocs_kernel(qp_ref, kp_ref, bx1_ref, by1_ref, bx2_ref, by2_ref, ab_ref,
                  base_ref, tib_ref, out_ref, ti_ref, cnt_ref):
    i = pl.program_id(0)
    bk = out_ref.shape[0]
    n = out_ref.shape[2]

    row8 = jax.lax.broadcasted_iota(jnp.int32, (8, 1), 0)
    r3 = row8 & 3
    is0 = r3 == 0
    is13 = (r3 & 1) == 1
    row4 = jax.lax.broadcasted_iota(jnp.int32, (4, 1), 0)
    is_k_row = (row4 & 1) == 0

    ch = 512
    full = (n // ch) * ch
    offs = [(o, ch) for o in range(0, full, ch)]
    if n % ch:
        offs.append((full, n % ch))

    acc = jnp.zeros((8, ch), jnp.int32)
    c_tail = jnp.zeros((), jnp.int32)
    for o, w in offs:
        bx1 = bx1_ref[:, pl.ds(o, w)]
        by1 = by1_ref[:, pl.ds(o, w)]
        bx2 = bx2_ref[:, pl.ds(o, w)]
        by2 = by2_ref[:, pl.ds(o, w)]
        ab = ab_ref[:, pl.ds(o, w)]
        base = base_ref[:, pl.ds(o, w)]
        tib = tib_ref[:, pl.ds(o, w)]
        for p in range(bk // 2):
            qx1 = qp_ref[p, :, 0:1]
            qy1 = qp_ref[p, :, 1:2]
            qx2 = qp_ref[p, :, 2:3]
            qy2 = qp_ref[p, :, 3:4]
            aq = qp_ref[p, :, 4:5]
            s2 = qp_ref[p, :, 5:6]

            iw = jnp.minimum(bx2, qx2) - jnp.maximum(bx1, qx1)
            ih = jnp.minimum(by2, qy2) - jnp.maximum(by1, qy1)
            inter = iw * ih
            iou = inter / ((ab + aq) - inter)
            valid = jnp.minimum(iw, ih) > 0.0

            f02 = jnp.where(valid, jnp.where(is0, iou, s2), -10.0)
            out_ref[pl.ds(2 * p, 2), :, pl.ds(o, w)] = jnp.where(
                is13, base, f02).reshape(2, 4, w)

            kv = kp_ref[p, :, 0:1]
            ti_ref[pl.ds(2 * p, 2), :, pl.ds(o, w)] = jnp.where(
                is_k_row, kv, tib).reshape(2, 2, w)

            ones = jnp.where(valid, 1, 0)
            if w == ch:
                acc = acc + ones
            else:
                c_tail = c_tail + jnp.sum(ones)

    c = (jnp.sum(acc) + c_tail) >> 2

    @pl.when(i == 0)
    def _init():
        cnt_ref[0, 0] = 0

    cnt_ref[0, 0] += c


def _rep8(x):
    return jnp.broadcast_to(x[None, :], (8, x.shape[0]))


def kernel(boxes, query_boxes, scores_3d, scores_2d, dis_to_lidar_3d):
    n = boxes.shape[0]
    k = query_boxes.shape[0]
    bk = 8

    b = boxes
    area_b = (b[:, 2] - b[:, 0]) * (b[:, 3] - b[:, 1])
    bx1 = _rep8(b[:, 0])
    by1 = _rep8(b[:, 1])
    bx2 = _rep8(b[:, 2])
    by2 = _rep8(b[:, 3])
    ab = _rep8(area_b)
    zeros = jnp.zeros((n,), jnp.float32)
    base = jnp.concatenate([
        jnp.stack([zeros, scores_3d[:, 0], zeros, dis_to_lidar_3d[:, 0]], 0)
    ] * 2, 0)
    nio = jnp.arange(n, dtype=jnp.int32)
    tib = jnp.stack([jnp.zeros((n,), jnp.int32), nio,
                     jnp.zeros((n,), jnp.int32), nio], 0)

    area_q = (query_boxes[:, 2] - query_boxes[:, 0]) * (
        query_boxes[:, 3] - query_boxes[:, 1])
    qcols = jnp.concatenate(
        [query_boxes, area_q[:, None], scores_2d, jnp.zeros((k, 2), jnp.float32)],
        axis=1)
    qpair = jnp.repeat(qcols, 4, axis=0).reshape(k // 2, 8, 8)
    kcol = jnp.repeat(jnp.arange(k, dtype=jnp.int32), 2).reshape(k // 2, 4, 1)

    grid = k // bk
    cvec = lambda nrows: pl.BlockSpec((nrows, n), lambda i: (0, 0))
    out, ti, cnt = pl.pallas_call(
        _clocs_kernel,
        grid=(grid,),
        in_specs=[
            pl.BlockSpec((bk // 2, 8, 8), lambda i: (i, 0, 0)),
            pl.BlockSpec((bk // 2, 4, 1), lambda i: (i, 0, 0)),
            cvec(8), cvec(8), cvec(8), cvec(8), cvec(8), cvec(8),
            pl.BlockSpec((4, n), lambda i: (0, 0)),
        ],
        out_specs=[
            pl.BlockSpec((bk, 4, n), lambda i: (i, 0, 0)),
            pl.BlockSpec((bk, 2, n), lambda i: (i, 0, 0)),
            pl.BlockSpec(memory_space=pltpu.SMEM, block_shape=(1, 1),
                         index_map=lambda i: (0, 0)),
        ],
        out_shape=[
            jax.ShapeDtypeStruct((k, 4, n), jnp.float32),
            jax.ShapeDtypeStruct((k, 2, n), jnp.int32),
            jax.ShapeDtypeStruct((1, 1), jnp.int32),
        ],
    )(qpair, kcol, bx1, by1, bx2, by2, ab, base, tib)

    overlaps = jnp.transpose(out, (0, 2, 1))
    tensor_index = jnp.transpose(ti, (0, 2, 1))
    return overlaps, tensor_index, cnt[0, 0]
